# Initial kernel scaffold; baseline (speedup 1.0000x reference)
import jax
import jax.numpy as jnp
from jax.experimental import pallas as pl


def kernel(x, edge_index, W_z, b_z, Wl_z, bl_z, W_r, b_r, Wl_r, bl_r, W_h, b_h, Wl_h, bl_h, att, W_out, b_out):
    raise NotImplementedError("write your pallas kernel here")


# speedup vs baseline: 9.9117x; 9.9117x over previous
import jax
import jax.numpy as jnp
from jax import lax
from jax.experimental import pallas as pl
from jax.experimental.pallas import tpu as pltpu
from jax.experimental.pallas import tpu_sc as plsc

N = 100000
E = 1600000
F = 32
P = 12
O = 32
D = F * P

N_PAD = 102400
K = 2560
NCHUNK = N_PAD // K
SW = 3
ACCR = (K + 8) * SW

NC, NS = 2, 16
OUT_STRIPE = K * SW // NS
E_PAD = 1638400
EPC = E_PAD // NC
EPT_DEG = EPC // NS
DEG_BATCH = 128
DEG_ITERS = EPT_DEG // DEG_BATCH
DEG_STRIPE = N_PAD // NS

EPT = E_PAD // NS
SEG = 4096
NSEG = EPT // SEG
SEG_VREGS = SEG // 16
GB = 112
ZR = 48


def _mesh():
    return plsc.VectorSubcoreMesh(core_axis_name="c", subcore_axis_name="s")


def _deg_body(cols_hbm, deg_out, idx_ref, ones_ref, zbuf, degsh):
    c = lax.axis_index("c")
    s = lax.axis_index("s")
    for u in range(DEG_BATCH // 16):
        ones_ref[pl.ds(u * 16, 16)] = jnp.ones((16,), jnp.float32)

    def zb(i, _):
        zbuf[pl.ds(i * 16, 16)] = jnp.zeros((16,), jnp.float32)
        return 0

    lax.fori_loop(0, DEG_STRIPE // 16, zb, 0)
    pltpu.sync_copy(zbuf, degsh.at[pl.ds(s * DEG_STRIPE, DEG_STRIPE)])
    plsc.subcore_barrier()

    base = c * EPC + s * EPT_DEG

    def body(i, _):
        pltpu.sync_copy(cols_hbm.at[pl.ds(base + i * DEG_BATCH, DEG_BATCH)],
                        idx_ref)
        pltpu.sync_copy(ones_ref, degsh.at[idx_ref], add=True)
        return 0

    lax.fori_loop(0, DEG_ITERS, body, 0)
    plsc.subcore_barrier()
    pltpu.sync_copy(degsh.at[pl.ds(s * DEG_STRIPE, DEG_STRIPE)],
                    deg_out.at[c, pl.ds(s * DEG_STRIPE, DEG_STRIPE)])


def _deg_call(cols_pad):
    return pl.kernel(
        _deg_body,
        out_type=jax.ShapeDtypeStruct((NC, N_PAD), jnp.float32),
        mesh=_mesh(),
        scratch_types=[
            pltpu.VMEM((DEG_BATCH,), jnp.int32),
            pltpu.VMEM((DEG_BATCH,), jnp.float32),
            pltpu.VMEM((DEG_STRIPE,), jnp.float32),
            pltpu.VMEM_SHARED((N_PAD,), jnp.float32),
        ],
        compiler_params=pltpu.CompilerParams(needs_layout_passes=False),
    )(cols_pad)


def _agg_body(rows_hbm, cols_hbm, y_hbm, t_hbm,
              ebuf_r, ebuf_c, crows, ccols, gidx, glist, slist, stage, zbuf,
              accsh):
    c = lax.axis_index("c")
    s = lax.axis_index("s")
    tile_base = s * EPT

    def zf(i, _):
        r = i // 8
        q = i % 8
        zbuf[r, pl.ds(q * 16, 16)] = jnp.zeros((16,), jnp.float32)
        return 0

    lax.fori_loop(0, ZR * 8, zf, 0)

    def chunk_body(ci, _):
        chunk = c + NC * ci
        cbase = chunk * K

        def zcp(i, _):
            pltpu.sync_copy(zbuf, accsh.at[pl.ds(s * OUT_STRIPE + i * ZR, ZR)])
            return 0

        lax.fori_loop(0, OUT_STRIPE // ZR, zcp, 0)
        plsc.subcore_barrier()

        def seg_body(si, _):
            ebase = tile_base + si * SEG
            pltpu.sync_copy(rows_hbm.at[pl.ds(ebase, SEG)], ebuf_r)
            pltpu.sync_copy(cols_hbm.at[pl.ds(ebase, SEG)], ebuf_c)

            def scan_body(j, cur):
                r16 = ebuf_r[pl.ds(j * 16, 16)]
                c16 = ebuf_c[pl.ds(j * 16, 16)]
                rel = c16 - cbase
                m = (rel >= 0) & (rel < K)
                pre = plsc.cumsum(m.astype(jnp.int32))
                idx = jnp.maximum(cur + pre - 1, 0)
                plsc.store_scatter(crows, [idx], r16, mask=m)
                plsc.store_scatter(ccols, [idx], rel, mask=m)
                return cur + pre[15]

            cur = lax.fori_loop(0, SEG_VREGS, scan_body, jnp.int32(0))
            for u in range(GB // 16):
                crows[pl.ds(cur + u * 16, 16)] = jnp.zeros((16,), jnp.int32)
                ccols[pl.ds(cur + u * 16, 16)] = jnp.full((16,), K, jnp.int32)
            nb = (cur + GB - 1) // GB

            def flush(bi, _):
                bo = bi * GB

                def bld(u, _):
                    rv = crows[pl.ds(bo + u * 16, 16)] * SW
                    cv = ccols[pl.ds(bo + u * 16, 16)] * SW
                    pos0 = (u * 16) * SW
                    ii = lax.iota(jnp.int32, 16) * SW
                    for q in range(SW):
                        plsc.store_scatter(glist, [ii + (pos0 + q)], rv + q)
                        plsc.store_scatter(slist, [ii + (pos0 + q)], cv + q)
                    return 0

                lax.fori_loop(0, GB // 16, bld, 0)
                pltpu.sync_copy(y_hbm.at[glist], stage)
                pltpu.sync_copy(stage, accsh.at[slist], add=True)
                return 0

            lax.fori_loop(0, nb, flush, 0)
            return 0

        lax.fori_loop(0, NSEG, seg_body, 0)
        plsc.subcore_barrier()
        o0 = s * OUT_STRIPE
        pltpu.sync_copy(accsh.at[pl.ds(o0, OUT_STRIPE)],
                        t_hbm.at[pl.ds(cbase * SW + o0, OUT_STRIPE)])
        plsc.subcore_barrier()
        return 0

    lax.fori_loop(0, NCHUNK // NC, chunk_body, 0)


def _agg_call(rows_pad, cols_pad, y_hw):
    return pl.kernel(
        _agg_body,
        out_type=jax.ShapeDtypeStruct((N_PAD * SW, 128), jnp.float32),
        mesh=_mesh(),
        scratch_types=[
            pltpu.VMEM((SEG,), jnp.int32),
            pltpu.VMEM((SEG,), jnp.int32),
            pltpu.VMEM((SEG + 2 * GB,), jnp.int32),
            pltpu.VMEM((SEG + 2 * GB,), jnp.int32),
            pltpu.VMEM((GB,), jnp.int32),
            pltpu.VMEM((GB * SW,), jnp.int32),
            pltpu.VMEM((GB * SW,), jnp.int32),
            pltpu.VMEM((GB * SW, 128), jnp.float32),
            pltpu.VMEM((ZR, 128), jnp.float32),
            pltpu.VMEM_SHARED((ACCR, 128), jnp.float32),
        ],
        compiler_params=pltpu.CompilerParams(needs_layout_passes=False),
    )(rows_pad, cols_pad, y_hw)


def _scale_body(deg_ref, x_ref, y_ref, dis_ref):
    d = deg_ref[:, 0:1] + deg_ref[:, 1:2] + 1.0
    dis = lax.rsqrt(d)
    y_ref[...] = x_ref[...] * dis
    dis_ref[...] = dis


def _scale_call(deg_t, x2):
    blk = 512
    return pl.pallas_call(
        _scale_body,
        grid=(N_PAD // blk,),
        in_specs=[
            pl.BlockSpec((blk, 2), lambda i: (i, 0)),
            pl.BlockSpec((blk, D), lambda i: (i, 0)),
        ],
        out_specs=[
            pl.BlockSpec((blk, D), lambda i: (i, 0)),
            pl.BlockSpec((blk, 1), lambda i: (i, 0)),
        ],
        out_shape=[
            jax.ShapeDtypeStruct((N_PAD, D), jnp.float32),
            jax.ShapeDtypeStruct((N_PAD, 1), jnp.float32),
        ],
    )(deg_t, x2)


def _dense_body(t_ref, y_ref, dis_ref, wz_ref, wlz_ref, bz_ref, blz_ref,
                wh_ref, wlh_ref, bh_ref, blh_ref, att_ref, wout_ref,
                bout_ref, out_ref):
    S = (t_ref[...] + y_ref[...]) * dis_ref[...]
    Wlz1 = wlz_ref[...][:, :O]
    Wlh1 = wlh_ref[...][:, :O]
    Mz = jnp.dot(Wlz1, wz_ref[...])
    Mh = jnp.dot(Wlh1, wh_ref[...])
    ct = (((1,), (1,)), ((), ()))
    bz_eff = lax.dot_general(bz_ref[...], Wlz1, ct) + blz_ref[...]
    bh_eff = lax.dot_general(bh_ref[...], Wlh1, ct) + blh_ref[...]

    a = att_ref[...]
    e = jnp.exp(a - jnp.max(a))
    pcol = e / jnp.sum(e)

    blk = S.shape[0]
    Hacc = jnp.zeros((blk, O), jnp.float32)
    for t in range(P):
        St = S[:, t * F:(t + 1) * F]
        Az = lax.dot_general(St, Mz, ct) + bz_eff
        Z = 1.0 / (1.0 + jnp.exp(-Az))
        Ah = lax.dot_general(St, Mh, ct) + bh_eff
        Ht = jnp.tanh(Ah)
        Hacc = Hacc + pcol[t:t + 1, 0:1] * ((1.0 - Z) * Ht)
    Hr = jnp.maximum(Hacc, 0.0)
    out_ref[...] = lax.dot_general(Hr, wout_ref[...], ct) + bout_ref[...]


def _dense_call(t_agg, y, dis, W_z, Wl_z, b_z, bl_z, W_h, Wl_h, b_h, bl_h,
                att, W_out, b_out):
    blk = 512

    def whole(shape):
        nd = len(shape)
        return pl.BlockSpec(shape, lambda i, _nd=nd: (0,) * _nd)

    return pl.pallas_call(
        _dense_body,
        grid=(N_PAD // blk,),
        in_specs=[
            pl.BlockSpec((blk, D), lambda i: (i, 0)),
            pl.BlockSpec((blk, D), lambda i: (i, 0)),
            pl.BlockSpec((blk, 1), lambda i: (i, 0)),
            whole((O, F)), whole((O, 2 * O)), whole((1, O)), whole((1, O)),
            whole((O, F)), whole((O, 2 * O)), whole((1, O)), whole((1, O)),
            whole((P, 1)), whole((P, O)), whole((1, P)),
        ],
        out_specs=pl.BlockSpec((blk, P), lambda i: (i, 0)),
        out_shape=jax.ShapeDtypeStruct((N_PAD, P), jnp.float32),
    )(t_agg, y, dis, W_z, Wl_z, b_z, bl_z, W_h, Wl_h, b_h, bl_h, att, W_out,
      b_out)


def kernel(x, edge_index, W_z, b_z, Wl_z, bl_z, W_r, b_r, Wl_r, bl_r,
           W_h, b_h, Wl_h, bl_h, att, W_out, b_out):
    del W_r, b_r, Wl_r, bl_r
    row = edge_index[0]
    col = edge_index[1]
    rows_pad = jnp.concatenate(
        [row, jnp.zeros((E_PAD - E,), jnp.int32)])
    cols_pad = jnp.concatenate(
        [col, jnp.full((E_PAD - E,), N_PAD - 1, jnp.int32)])

    x2 = jnp.transpose(x, (0, 2, 1)).reshape(N, D)
    x2 = jnp.pad(x2, ((0, N_PAD - N), (0, 0)))

    deg2 = _deg_call(cols_pad)
    y, dis = _scale_call(deg2.T, x2)
    t_agg = _agg_call(rows_pad, cols_pad,
                      y.reshape(N_PAD * SW, 128)).reshape(N_PAD, D)
    out = _dense_call(
        t_agg, y, dis, W_z, Wl_z, b_z.reshape(1, O), bl_z.reshape(1, O),
        W_h, Wl_h, b_h.reshape(1, O), bl_h.reshape(1, O),
        att.reshape(P, 1), W_out, b_out.reshape(1, P))
    return out[:N]


# speedup vs baseline: 12.6443x; 1.2757x over previous
import jax
import jax.numpy as jnp
from jax import lax
from jax.experimental import pallas as pl
from jax.experimental.pallas import tpu as pltpu
from jax.experimental.pallas import tpu_sc as plsc

N = 100000
E = 1600000
F = 32
P = 12
O = 32
D = F * P

N_PAD = 102400
K = 2560
NCHUNK = N_PAD // K
SW = 3
ACCR = (K + 8) * SW

NC, NS = 2, 16
OUT_STRIPE = K * SW // NS
E_PAD = 1638400
EPC = E_PAD // NC
EPT_DEG = EPC // NS
DEG_BATCH = 128
DEG_ITERS = EPT_DEG // DEG_BATCH
DEG_STRIPE = N_PAD // NS

EPT = E_PAD // NS
SEG = 4096
NSEG = EPT // SEG
SEG_VREGS = SEG // 16
GB = 64
ZR = 16


def _mesh():
    return plsc.VectorSubcoreMesh(core_axis_name="c", subcore_axis_name="s")


def _deg_body(cols_hbm, deg_out, idx_ref, ones_ref, zbuf, degsh):
    c = lax.axis_index("c")
    s = lax.axis_index("s")
    for u in range(DEG_BATCH // 16):
        ones_ref[pl.ds(u * 16, 16)] = jnp.ones((16,), jnp.float32)

    def zb(i, _):
        zbuf[pl.ds(i * 16, 16)] = jnp.zeros((16,), jnp.float32)
        return 0

    lax.fori_loop(0, DEG_STRIPE // 16, zb, 0)
    pltpu.sync_copy(zbuf, degsh.at[pl.ds(s * DEG_STRIPE, DEG_STRIPE)])
    plsc.subcore_barrier()

    base = c * EPC + s * EPT_DEG

    def body(i, _):
        pltpu.sync_copy(cols_hbm.at[pl.ds(base + i * DEG_BATCH, DEG_BATCH)],
                        idx_ref)
        pltpu.sync_copy(ones_ref, degsh.at[idx_ref], add=True)
        return 0

    lax.fori_loop(0, DEG_ITERS, body, 0)
    plsc.subcore_barrier()
    pltpu.sync_copy(degsh.at[pl.ds(s * DEG_STRIPE, DEG_STRIPE)],
                    deg_out.at[c, pl.ds(s * DEG_STRIPE, DEG_STRIPE)])


def _deg_call(cols_pad):
    return pl.kernel(
        _deg_body,
        out_type=jax.ShapeDtypeStruct((NC, N_PAD), jnp.float32),
        mesh=_mesh(),
        scratch_types=[
            pltpu.VMEM((DEG_BATCH,), jnp.int32),
            pltpu.VMEM((DEG_BATCH,), jnp.float32),
            pltpu.VMEM((DEG_STRIPE,), jnp.float32),
            pltpu.VMEM_SHARED((N_PAD,), jnp.float32),
        ],
        compiler_params=pltpu.CompilerParams(needs_layout_passes=False),
    )(cols_pad)


def _agg_body(rows_hbm, cols_hbm, y_hbm, t_hbm,
              ebuf_r, ebuf_c, crows, ccols, glist, slist0, slist1,
              stage0, stage1, sem0, sem1, zbuf, accsh):
    c = lax.axis_index("c")
    s = lax.axis_index("s")
    tile_base = s * EPT

    def zf(i, _):
        r = i // 8
        q = i % 8
        zbuf[r, pl.ds(q * 16, 16)] = jnp.zeros((16,), jnp.float32)
        return 0

    lax.fori_loop(0, ZR * 8, zf, 0)

    def chunk_body(ci, _):
        chunk = c + NC * ci
        cbase = chunk * K

        def zcp(i, _):
            pltpu.sync_copy(zbuf, accsh.at[pl.ds(s * OUT_STRIPE + i * ZR, ZR)])
            return 0

        lax.fori_loop(0, OUT_STRIPE // ZR, zcp, 0)
        plsc.subcore_barrier()

        def do_flush(fc, bo, slist_b, stage_b, sem_b):
            @pl.when(fc >= 2)
            def _():
                pltpu.make_async_copy(stage_b, accsh.at[slist_b],
                                      sem_b).wait()

            def bld(u, _):
                rv = crows[pl.ds(bo + u * 16, 16)] * SW
                cv = ccols[pl.ds(bo + u * 16, 16)] * SW
                pos0 = (u * 16) * SW
                ii = lax.iota(jnp.int32, 16) * SW
                for q in range(SW):
                    plsc.store_scatter(glist, [ii + (pos0 + q)], rv + q)
                    plsc.store_scatter(slist_b, [ii + (pos0 + q)], cv + q)
                return 0

            lax.fori_loop(0, GB // 16, bld, 0)
            pltpu.sync_copy(y_hbm.at[glist], stage_b)
            pltpu.make_async_copy(stage_b, accsh.at[slist_b],
                                  sem_b).start(add=True)

        def seg_body(si, fc):
            ebase = tile_base + si * SEG
            pltpu.sync_copy(rows_hbm.at[pl.ds(ebase, SEG)], ebuf_r)
            pltpu.sync_copy(cols_hbm.at[pl.ds(ebase, SEG)], ebuf_c)

            def scan_body(j, cur):
                r16 = ebuf_r[pl.ds(j * 16, 16)]
                c16 = ebuf_c[pl.ds(j * 16, 16)]
                rel = c16 - cbase
                m = (rel >= 0) & (rel < K)
                pre = plsc.cumsum(m.astype(jnp.int32))
                idx = jnp.maximum(cur + pre - 1, 0)
                plsc.store_scatter(crows, [idx], r16, mask=m)
                plsc.store_scatter(ccols, [idx], rel, mask=m)
                return cur + pre[15]

            cur = lax.fori_loop(0, SEG_VREGS, scan_body, jnp.int32(0))
            for u in range(GB // 16):
                crows[pl.ds(cur + u * 16, 16)] = jnp.zeros((16,), jnp.int32)
                ccols[pl.ds(cur + u * 16, 16)] = jnp.full((16,), K, jnp.int32)
            nb = (cur + GB - 1) // GB

            def flush(bi, fc2):
                bo = bi * GB

                @pl.when(fc2 % 2 == 0)
                def _():
                    do_flush(fc2, bo, slist0, stage0, sem0)

                @pl.when(fc2 % 2 == 1)
                def _():
                    do_flush(fc2, bo, slist1, stage1, sem1)

                return fc2 + 1

            return lax.fori_loop(0, nb, flush, fc)

        fc = lax.fori_loop(0, NSEG, seg_body, jnp.int32(0))

        @pl.when((fc >= 1) & (fc % 2 == 1))
        def _():
            pltpu.make_async_copy(stage0, accsh.at[slist0], sem0).wait()

        @pl.when((fc >= 1) & (fc % 2 == 0))
        def _():
            pltpu.make_async_copy(stage1, accsh.at[slist1], sem1).wait()

        @pl.when((fc >= 2) & (fc % 2 == 1))
        def _():
            pltpu.make_async_copy(stage1, accsh.at[slist1], sem1).wait()

        @pl.when((fc >= 2) & (fc % 2 == 0))
        def _():
            pltpu.make_async_copy(stage0, accsh.at[slist0], sem0).wait()

        plsc.subcore_barrier()
        o0 = s * OUT_STRIPE
        pltpu.sync_copy(accsh.at[pl.ds(o0, OUT_STRIPE)],
                        t_hbm.at[pl.ds(cbase * SW + o0, OUT_STRIPE)])
        plsc.subcore_barrier()
        return 0

    lax.fori_loop(0, NCHUNK // NC, chunk_body, 0)


def _agg_call(rows_pad, cols_pad, y_hw):
    return pl.kernel(
        _agg_body,
        out_type=jax.ShapeDtypeStruct((N_PAD * SW, 128), jnp.float32),
        mesh=_mesh(),
        scratch_types=[
            pltpu.VMEM((SEG,), jnp.int32),
            pltpu.VMEM((SEG,), jnp.int32),
            pltpu.VMEM((SEG + 2 * GB,), jnp.int32),
            pltpu.VMEM((SEG + 2 * GB,), jnp.int32),
            pltpu.VMEM((GB * SW,), jnp.int32),
            pltpu.VMEM((GB * SW,), jnp.int32),
            pltpu.VMEM((GB * SW,), jnp.int32),
            pltpu.VMEM((GB * SW, 128), jnp.float32),
            pltpu.VMEM((GB * SW, 128), jnp.float32),
            pltpu.SemaphoreType.DMA,
            pltpu.SemaphoreType.DMA,
            pltpu.VMEM((ZR, 128), jnp.float32),
            pltpu.VMEM_SHARED((ACCR, 128), jnp.float32),
        ],
        compiler_params=pltpu.CompilerParams(needs_layout_passes=False),
    )(rows_pad, cols_pad, y_hw)


def _scale_body(deg_ref, x_ref, y_ref, dis_ref):
    d = deg_ref[:, 0:1] + deg_ref[:, 1:2] + 1.0
    dis = lax.rsqrt(d)
    y_ref[...] = x_ref[...] * dis
    dis_ref[...] = dis


def _scale_call(deg_t, x2):
    blk = 512
    return pl.pallas_call(
        _scale_body,
        grid=(N_PAD // blk,),
        in_specs=[
            pl.BlockSpec((blk, 2), lambda i: (i, 0)),
            pl.BlockSpec((blk, D), lambda i: (i, 0)),
        ],
        out_specs=[
            pl.BlockSpec((blk, D), lambda i: (i, 0)),
            pl.BlockSpec((blk, 1), lambda i: (i, 0)),
        ],
        out_shape=[
            jax.ShapeDtypeStruct((N_PAD, D), jnp.float32),
            jax.ShapeDtypeStruct((N_PAD, 1), jnp.float32),
        ],
    )(deg_t, x2)


def _dense_body(t_ref, y_ref, dis_ref, wz_ref, wlz_ref, bz_ref, blz_ref,
                wh_ref, wlh_ref, bh_ref, blh_ref, att_ref, wout_ref,
                bout_ref, out_ref):
    S = (t_ref[...] + y_ref[...]) * dis_ref[...]
    Wlz1 = wlz_ref[...][:, :O]
    Wlh1 = wlh_ref[...][:, :O]
    Mz = jnp.dot(Wlz1, wz_ref[...])
    Mh = jnp.dot(Wlh1, wh_ref[...])
    ct = (((1,), (1,)), ((), ()))
    bz_eff = lax.dot_general(bz_ref[...], Wlz1, ct) + blz_ref[...]
    bh_eff = lax.dot_general(bh_ref[...], Wlh1, ct) + blh_ref[...]

    a = att_ref[...]
    e = jnp.exp(a - jnp.max(a))
    pcol = e / jnp.sum(e)

    blk = S.shape[0]
    Hacc = jnp.zeros((blk, O), jnp.float32)
    for t in range(P):
        St = S[:, t * F:(t + 1) * F]
        Az = lax.dot_general(St, Mz, ct) + bz_eff
        Z = 1.0 / (1.0 + jnp.exp(-Az))
        Ah = lax.dot_general(St, Mh, ct) + bh_eff
        Ht = jnp.tanh(Ah)
        Hacc = Hacc + pcol[t:t + 1, 0:1] * ((1.0 - Z) * Ht)
    Hr = jnp.maximum(Hacc, 0.0)
    out_ref[...] = lax.dot_general(Hr, wout_ref[...], ct) + bout_ref[...]


def _dense_call(t_agg, y, dis, W_z, Wl_z, b_z, bl_z, W_h, Wl_h, b_h, bl_h,
                att, W_out, b_out):
    blk = 512

    def whole(shape):
        nd = len(shape)
        return pl.BlockSpec(shape, lambda i, _nd=nd: (0,) * _nd)

    return pl.pallas_call(
        _dense_body,
        grid=(N_PAD // blk,),
        in_specs=[
            pl.BlockSpec((blk, D), lambda i: (i, 0)),
            pl.BlockSpec((blk, D), lambda i: (i, 0)),
            pl.BlockSpec((blk, 1), lambda i: (i, 0)),
            whole((O, F)), whole((O, 2 * O)), whole((1, O)), whole((1, O)),
            whole((O, F)), whole((O, 2 * O)), whole((1, O)), whole((1, O)),
            whole((P, 1)), whole((P, O)), whole((1, P)),
        ],
        out_specs=pl.BlockSpec((blk, P), lambda i: (i, 0)),
        out_shape=jax.ShapeDtypeStruct((N_PAD, P), jnp.float32),
    )(t_agg, y, dis, W_z, Wl_z, b_z, bl_z, W_h, Wl_h, b_h, bl_h, att, W_out,
      b_out)


def kernel(x, edge_index, W_z, b_z, Wl_z, bl_z, W_r, b_r, Wl_r, bl_r,
           W_h, b_h, Wl_h, bl_h, att, W_out, b_out):
    del W_r, b_r, Wl_r, bl_r
    row = edge_index[0]
    col = edge_index[1]
    rows_pad = jnp.concatenate(
        [row, jnp.zeros((E_PAD - E,), jnp.int32)])
    cols_pad = jnp.concatenate(
        [col, jnp.full((E_PAD - E,), N_PAD - 1, jnp.int32)])

    x2 = jnp.transpose(x, (0, 2, 1)).reshape(N, D)
    x2 = jnp.pad(x2, ((0, N_PAD - N), (0, 0)))

    deg2 = _deg_call(cols_pad)
    y, dis = _scale_call(deg2.T, x2)
    t_agg = _agg_call(rows_pad, cols_pad,
                      y.reshape(N_PAD * SW, 128)).reshape(N_PAD, D)
    out = _dense_call(
        t_agg, y, dis, W_z, Wl_z, b_z.reshape(1, O), bl_z.reshape(1, O),
        W_h, Wl_h, b_h.reshape(1, O), bl_h.reshape(1, O),
        att.reshape(P, 1), W_out, b_out.reshape(1, P))
    return out[:N]
